# Initial kernel scaffold; baseline (speedup 1.0000x reference)
import jax
import jax.numpy as jnp
from jax.experimental import pallas as pl


def kernel(x, grid_values):
    raise NotImplementedError("write your pallas kernel here")


# speedup vs baseline: 11.1951x; 11.1951x over previous
import jax
import jax.numpy as jnp
import numpy as np
from jax import lax
from jax.experimental import pallas as pl
from jax.experimental.pallas import tpu as pltpu
from jax.experimental.pallas import tpu_sc as plsc

_HASH_SIZE = 2 ** 19
_K1 = int(np.int32(np.uint32(2654435761)))
_K2 = int(np.int32(np.uint32(805459861)))
_NF = 2
_NPTS = 65536


def _level_grid_sizes():
    desired = 1.0 + 2.0 * np.log2(2048 / 16)
    n = int(np.round(desired))
    return [int(g) for g in np.round(np.geomspace(16, 2048, n)).astype(np.int32)]


_GS = _level_grid_sizes()
_IS_DENSE = [g ** 3 <= _HASH_SIZE for g in _GS]
_NLVL = len(_GS)
_NOUT = 2 * _NLVL

_NC, _NS = 2, 16
_NW = _NC * _NS
_PW = _NPTS // _NW
_CHUNK = 1024
_NCK = _PW // _CHUNK
_NGRP = _CHUNK // 16
_NSTR = 2 * _NGRP


def _body(x_ref, *rest):
    tabs = rest[:_NLVL]
    out_ref = rest[_NLVL]
    (xbuf, ybuf, zbuf, idxb, wbuf, rows, outb, sem) = rest[_NLVL + 1:]

    cid = lax.axis_index("c")
    sid = lax.axis_index("s")
    wid = sid * _NC + cid
    base = wid * _PW

    iota16 = lax.iota(jnp.int32, 16)

    pltpu.sync_copy(x_ref.at[0, pl.ds(base, _PW)], xbuf)
    pltpu.sync_copy(x_ref.at[1, pl.ds(base, _PW)], ybuf)
    pltpu.sync_copy(x_ref.at[2, pl.ds(base, _PW)], zbuf)

    def pass0(o, carry):
        s = pl.ds(o * 16, 16)
        xbuf[s] = (xbuf[s] + 2.0) * 0.25
        ybuf[s] = (ybuf[s] + 2.0) * 0.25
        zbuf[s] = (zbuf[s] + 2.0) * 0.25
        return carry

    lax.fori_loop(0, _PW // 16, pass0, 0)

    for lvl in range(_NLVL):
        gs = _GS[lvl]
        dense = _IS_DENSE[lvl]
        tab = tabs[lvl]
        gsf = float(gs)

        def chunk_body(ck, carry, tab=tab, gs=gs, dense=dense, lvl=lvl, gsf=gsf):
            cbase = ck * _CHUNK

            def pass1(o, c1):
                p0 = cbase + o * 16
                vx = xbuf[pl.ds(p0, 16)] * gsf
                vy = ybuf[pl.ds(p0, 16)] * gsf
                vz = zbuf[pl.ds(p0, 16)] * gsf
                terms = []
                wpairs = []
                if dense:
                    for v, mulc in ((vz, 2 * gs * gs), (vy, 2 * gs), (vx, 2)):
                        lc = v - 0.5
                        it = lc.astype(jnp.int32)
                        ft = it.astype(jnp.float32)
                        it = jnp.where(ft > lc, it - 1, it)
                        ft = it.astype(jnp.float32)
                        t = lc - ft
                        i0 = jnp.maximum(it, 0) * mulc
                        i1 = jnp.minimum(it + 1, gs - 1) * mulc
                        terms.append((i0, i1))
                        wpairs.append((1.0 - t, t))
                else:
                    for v, kc in ((vx, 1), (vy, _K1), (vz, _K2)):
                        iv = v.astype(jnp.int32)
                        fv = iv.astype(jnp.float32)
                        t = v - fv
                        h0 = iv * kc if kc != 1 else iv
                        h1 = h0 + kc
                        terms.append((h0, h1))
                        wpairs.append((1.0 - t, t))
                for c in range(8):
                    a, b, d = (c >> 2) & 1, (c >> 1) & 1, c & 1
                    if dense:
                        e0 = terms[0][a] + terms[1][b] + terms[2][d]
                    else:
                        h = ((terms[0][a] ^ terms[1][b]) ^ terms[2][d]) & (_HASH_SIZE - 1)
                        e0 = h * 2
                    w = (wpairs[0][a] * wpairs[1][b]) * wpairs[2][d]
                    idxb[o, pl.ds(c * 16, 16)] = e0
                    idxb[_NGRP + o, pl.ds(c * 16, 16)] = e0 + 1
                    wbuf[o, pl.ds(c * 16, 16)] = w
                return c1

            lax.fori_loop(0, _NGRP, pass1, 0)

            def issue(j, c1):
                pltpu.make_async_copy(
                    tab.at[idxb.at[j]],
                    rows.at[pl.ds(j * 128, 128)],
                    sem,
                ).start()
                return c1

            lax.fori_loop(0, _NSTR, issue, 0)

            def drain(j, c1):
                pltpu.make_async_copy(
                    tab.at[idxb.at[j]],
                    rows.at[pl.ds(j * 128, 128)],
                    sem,
                ).wait()
                return c1

            lax.fori_loop(0, _NSTR, drain, 0)

            half = _NGRP * 128

            def pass2(o, c1):
                rb = o * 128
                acc0 = jnp.zeros((16,), jnp.float32)
                acc1 = jnp.zeros((16,), jnp.float32)
                for c in range(8):
                    g0 = rows[pl.ds(rb + c * 16, 16)]
                    g1 = rows[pl.ds(half + rb + c * 16, 16)]
                    wv = wbuf[o, pl.ds(c * 16, 16)]
                    acc0 = acc0 + wv * g0
                    acc1 = acc1 + wv * g1
                pid = cbase + o * 16 + iota16
                oid = pid * _NOUT + (2 * lvl)
                plsc.store_scatter(outb, [oid], acc0 * 10.0)
                plsc.store_scatter(outb, [oid + 1], acc1 * 10.0)
                return c1

            lax.fori_loop(0, _NGRP, pass2, 0)
            return carry

        lax.fori_loop(0, _NCK, chunk_body, 0)

    pltpu.sync_copy(outb, out_ref.at[pl.ds(base * _NOUT, _PW * _NOUT)])


def _make_kernel():
    mesh = plsc.VectorSubcoreMesh(
        core_axis_name="c", subcore_axis_name="s",
        num_cores=_NC, num_subcores=_NS)
    return pl.kernel(
        _body,
        out_type=jax.ShapeDtypeStruct((_NPTS * _NOUT,), jnp.float32),
        mesh=mesh,
        compiler_params=pltpu.CompilerParams(
            needs_layout_passes=False, use_tc_tiling_on_sc=False),
        scratch_types=[
            pltpu.VMEM((_PW,), jnp.float32),
            pltpu.VMEM((_PW,), jnp.float32),
            pltpu.VMEM((_PW,), jnp.float32),
            pltpu.VMEM((_NSTR, 128), jnp.int32),
            pltpu.VMEM((_NGRP, 128), jnp.float32),
            pltpu.VMEM((_NSTR * 128,), jnp.float32),
            pltpu.VMEM((_PW * _NOUT,), jnp.float32),
            pltpu.SemaphoreType.DMA,
        ],
    )


def kernel(x, grid_values):
    tabs = [tv.reshape(-1) for tv in grid_values]
    xt = x.T
    flat = _make_kernel()(xt, *tabs)
    return flat.reshape(_NPTS, _NOUT)


# speedup vs baseline: 52.0941x; 4.6533x over previous
import jax
import jax.numpy as jnp
import numpy as np
from jax import lax
from jax.experimental import pallas as pl
from jax.experimental.pallas import tpu as pltpu
from jax.experimental.pallas import tpu_sc as plsc

_HASH_SIZE = 2 ** 19
_K1 = int(np.int32(np.uint32(2654435761)))
_K2 = int(np.int32(np.uint32(805459861)))
_NF = 2
_NPTS = 65536


def _level_grid_sizes():
    desired = 1.0 + 2.0 * np.log2(2048 / 16)
    n = int(np.round(desired))
    return [int(g) for g in np.round(np.geomspace(16, 2048, n)).astype(np.int32)]


_GS = _level_grid_sizes()
_IS_DENSE = [g ** 3 <= _HASH_SIZE for g in _GS]
_NLVL = len(_GS)
_NOUT = 2 * _NLVL

_NC, _NS = 2, 16
_NW = _NC * _NS
_PW = _NPTS // _NW
_CHUNK = 1024
_NCK = _PW // _CHUNK
_NGRP = _CHUNK // 16


def _body(x_ref, *rest):
    tabs0 = rest[:_NLVL]
    tabs1 = rest[_NLVL:2 * _NLVL]
    out_ref = rest[2 * _NLVL]
    (xbuf, ybuf, zbuf, idxb, wbuf, rows, outb, sem) = rest[2 * _NLVL + 1:]

    cid = lax.axis_index("c")
    sid = lax.axis_index("s")
    wid = sid * _NC + cid
    base = wid * _PW

    iota16 = lax.iota(jnp.int32, 16)

    pltpu.sync_copy(x_ref.at[0, pl.ds(base, _PW)], xbuf)
    pltpu.sync_copy(x_ref.at[1, pl.ds(base, _PW)], ybuf)
    pltpu.sync_copy(x_ref.at[2, pl.ds(base, _PW)], zbuf)

    def pass0(o, carry):
        s = pl.ds(o * 16, 16)
        xbuf[s] = (xbuf[s] + 2.0) * 0.25
        ybuf[s] = (ybuf[s] + 2.0) * 0.25
        zbuf[s] = (zbuf[s] + 2.0) * 0.25
        return carry

    lax.fori_loop(0, _PW // 16, pass0, 0)

    half = _NGRP * 128

    for lvl in range(_NLVL):
        gs = _GS[lvl]
        dense = _IS_DENSE[lvl]
        t0 = tabs0[lvl]
        t1 = tabs1[lvl]
        gsf = float(gs)

        def chunk_body(ck, carry, t0=t0, t1=t1, gs=gs, dense=dense,
                       lvl=lvl, gsf=gsf):
            cbase = ck * _CHUNK

            def pass1(o, c1):
                p0 = cbase + o * 16
                vx = xbuf[pl.ds(p0, 16)] * gsf
                vy = ybuf[pl.ds(p0, 16)] * gsf
                vz = zbuf[pl.ds(p0, 16)] * gsf
                terms = []
                wpairs = []
                if dense:
                    for v, mulc in ((vz, gs * gs), (vy, gs), (vx, 1)):
                        lc = v - 0.5
                        it = lc.astype(jnp.int32)
                        ft = it.astype(jnp.float32)
                        it = jnp.where(ft > lc, it - 1, it)
                        ft = it.astype(jnp.float32)
                        t = lc - ft
                        i0 = jnp.maximum(it, 0) * mulc
                        i1 = jnp.minimum(it + 1, gs - 1) * mulc
                        terms.append((i0, i1))
                        wpairs.append((1.0 - t, t))
                else:
                    for v, kc in ((vx, 1), (vy, _K1), (vz, _K2)):
                        iv = v.astype(jnp.int32)
                        fv = iv.astype(jnp.float32)
                        t = v - fv
                        h0 = iv * kc if kc != 1 else iv
                        h1 = h0 + kc
                        terms.append((h0, h1))
                        wpairs.append((1.0 - t, t))
                for c in range(8):
                    a, b, d = (c >> 2) & 1, (c >> 1) & 1, c & 1
                    if dense:
                        row = terms[0][a] + terms[1][b] + terms[2][d]
                    else:
                        row = ((terms[0][a] ^ terms[1][b]) ^ terms[2][d]) & (_HASH_SIZE - 1)
                    w = (wpairs[0][a] * wpairs[1][b]) * wpairs[2][d]
                    idxb[o, pl.ds(c * 16, 16)] = row
                    wbuf[o, pl.ds(c * 16, 16)] = w
                return c1

            lax.fori_loop(0, _NGRP, pass1, 0)

            def issue(j, c1):
                pltpu.make_async_copy(
                    t0.at[idxb.at[j]], rows.at[pl.ds(j * 128, 128)], sem,
                ).start()
                pltpu.make_async_copy(
                    t1.at[idxb.at[j]], rows.at[pl.ds(half + j * 128, 128)], sem,
                ).start()
                return c1

            lax.fori_loop(0, _NGRP, issue, 0)

            def drain(j, c1):
                pltpu.make_async_copy(
                    t0.at[idxb.at[j]], rows.at[pl.ds(j * 128, 128)], sem,
                ).wait()
                pltpu.make_async_copy(
                    t1.at[idxb.at[j]], rows.at[pl.ds(half + j * 128, 128)], sem,
                ).wait()
                return c1

            lax.fori_loop(0, _NGRP, drain, 0)

            def pass2(o, c1):
                rb = o * 128
                acc0 = jnp.zeros((16,), jnp.float32)
                acc1 = jnp.zeros((16,), jnp.float32)
                for c in range(8):
                    g0 = rows[pl.ds(rb + c * 16, 16)]
                    g1 = rows[pl.ds(half + rb + c * 16, 16)]
                    wv = wbuf[o, pl.ds(c * 16, 16)]
                    acc0 = acc0 + wv * g0
                    acc1 = acc1 + wv * g1
                pid = cbase + o * 16 + iota16
                oid = pid * _NOUT + (2 * lvl)
                plsc.store_scatter(outb, [oid], acc0 * 10.0)
                plsc.store_scatter(outb, [oid + 1], acc1 * 10.0)
                return c1

            lax.fori_loop(0, _NGRP, pass2, 0)
            return carry

        lax.fori_loop(0, _NCK, chunk_body, 0)

    pltpu.sync_copy(outb, out_ref.at[pl.ds(base * _NOUT, _PW * _NOUT)])


def _make_kernel():
    mesh = plsc.VectorSubcoreMesh(
        core_axis_name="c", subcore_axis_name="s",
        num_cores=_NC, num_subcores=_NS)
    return pl.kernel(
        _body,
        out_type=jax.ShapeDtypeStruct((_NPTS * _NOUT,), jnp.float32),
        mesh=mesh,
        compiler_params=pltpu.CompilerParams(
            needs_layout_passes=False, use_tc_tiling_on_sc=False),
        scratch_types=[
            pltpu.VMEM((_PW,), jnp.float32),
            pltpu.VMEM((_PW,), jnp.float32),
            pltpu.VMEM((_PW,), jnp.float32),
            pltpu.VMEM((_NGRP, 128), jnp.int32),
            pltpu.VMEM((_NGRP, 128), jnp.float32),
            pltpu.VMEM((2 * _NGRP * 128,), jnp.float32),
            pltpu.VMEM((_PW * _NOUT,), jnp.float32),
            pltpu.SemaphoreType.DMA,
        ],
    )


def kernel(x, grid_values):
    tabs0 = []
    tabs1 = []
    for tv in grid_values:
        t2 = tv.reshape(-1, _NF)
        tabs0.append(t2[:, 0])
        tabs1.append(t2[:, 1])
    xt = x.T
    flat = _make_kernel()(xt, *tabs0, *tabs1)
    return flat.reshape(_NPTS, _NOUT)


# speedup vs baseline: 54.7615x; 1.0512x over previous
import jax
import jax.numpy as jnp
import numpy as np
from jax import lax
from jax.experimental import pallas as pl
from jax.experimental.pallas import tpu as pltpu
from jax.experimental.pallas import tpu_sc as plsc

_HASH_SIZE = 2 ** 19
_K1 = int(np.int32(np.uint32(2654435761)))
_K2 = int(np.int32(np.uint32(805459861)))
_NF = 2
_NPTS = 65536


def _level_grid_sizes():
    desired = 1.0 + 2.0 * np.log2(2048 / 16)
    n = int(np.round(desired))
    return [int(g) for g in np.round(np.geomspace(16, 2048, n)).astype(np.int32)]


_GS = _level_grid_sizes()
_IS_DENSE = [g ** 3 <= _HASH_SIZE for g in _GS]
_NLVL = len(_GS)
_NOUT = 2 * _NLVL

_NC, _NS = 2, 16
_NW = _NC * _NS
_PW = _NPTS // _NW
_CHUNK = 1024
_NCK = _PW // _CHUNK
_NGRP = _CHUNK // 16
_LIST = 512
_GPL = _LIST // 128
_NLIST = _NGRP // _GPL


def _body(x_ref, *rest):
    tabs0 = rest[:_NLVL]
    tabs1 = rest[_NLVL:2 * _NLVL]
    out_ref = rest[2 * _NLVL]
    (xbuf, ybuf, zbuf, idxb, wbuf, rows, outb, sem) = rest[2 * _NLVL + 1:]

    cid = lax.axis_index("c")
    sid = lax.axis_index("s")
    wid = sid * _NC + cid
    base = wid * _PW

    iota16 = lax.iota(jnp.int32, 16)

    pltpu.sync_copy(x_ref.at[0, pl.ds(base, _PW)], xbuf)
    pltpu.sync_copy(x_ref.at[1, pl.ds(base, _PW)], ybuf)
    pltpu.sync_copy(x_ref.at[2, pl.ds(base, _PW)], zbuf)

    def pass0(o, carry):
        s = pl.ds(o * 16, 16)
        xbuf[s] = (xbuf[s] + 2.0) * 0.25
        ybuf[s] = (ybuf[s] + 2.0) * 0.25
        zbuf[s] = (zbuf[s] + 2.0) * 0.25
        return carry

    lax.fori_loop(0, _PW // 16, pass0, 0)

    half = _NGRP * 128

    for lvl in range(_NLVL):
        gs = _GS[lvl]
        dense = _IS_DENSE[lvl]
        t0 = tabs0[lvl]
        t1 = tabs1[lvl]
        gsf = float(gs)

        def chunk_body(ck, carry, t0=t0, t1=t1, gs=gs, dense=dense,
                       lvl=lvl, gsf=gsf):
            cbase = ck * _CHUNK

            def pass1(o, c1):
                p0 = cbase + o * 16
                vx = xbuf[pl.ds(p0, 16)] * gsf
                vy = ybuf[pl.ds(p0, 16)] * gsf
                vz = zbuf[pl.ds(p0, 16)] * gsf
                terms = []
                wpairs = []
                if dense:
                    for v, mulc in ((vz, gs * gs), (vy, gs), (vx, 1)):
                        lc = v - 0.5
                        it = lc.astype(jnp.int32)
                        ft = it.astype(jnp.float32)
                        it = jnp.where(ft > lc, it - 1, it)
                        ft = it.astype(jnp.float32)
                        t = lc - ft
                        i0 = jnp.maximum(it, 0) * mulc
                        i1 = jnp.minimum(it + 1, gs - 1) * mulc
                        terms.append((i0, i1))
                        wpairs.append((1.0 - t, t))
                else:
                    for v, kc in ((vx, 1), (vy, _K1), (vz, _K2)):
                        iv = v.astype(jnp.int32)
                        fv = iv.astype(jnp.float32)
                        t = v - fv
                        h0 = iv * kc if kc != 1 else iv
                        h1 = h0 + kc
                        terms.append((h0, h1))
                        wpairs.append((1.0 - t, t))
                for c in range(8):
                    a, b, d = (c >> 2) & 1, (c >> 1) & 1, c & 1
                    if dense:
                        row = terms[0][a] + terms[1][b] + terms[2][d]
                    else:
                        row = ((terms[0][a] ^ terms[1][b]) ^ terms[2][d]) & (_HASH_SIZE - 1)
                    w = (wpairs[0][a] * wpairs[1][b]) * wpairs[2][d]
                    idxb[pl.ds(o * 128 + c * 16, 16)] = row
                    wbuf[o, pl.ds(c * 16, 16)] = w

                @pl.when(lax.rem(o, _GPL) == _GPL - 1)
                def _():
                    j = lax.div(o, _GPL)
                    s = pl.ds(j * _LIST, _LIST)
                    pltpu.make_async_copy(t0.at[idxb.at[s]], rows.at[s],
                                          sem).start()
                    pltpu.make_async_copy(
                        t1.at[idxb.at[s]],
                        rows.at[pl.ds(half + j * _LIST, _LIST)], sem).start()
                return c1

            lax.fori_loop(0, _NGRP, pass1, 0)

            def drain(j, c1):
                s = pl.ds(j * _LIST, _LIST)
                pltpu.make_async_copy(t0.at[idxb.at[s]], rows.at[s],
                                      sem).wait()
                pltpu.make_async_copy(
                    t1.at[idxb.at[s]],
                    rows.at[pl.ds(half + j * _LIST, _LIST)], sem).wait()
                return c1

            lax.fori_loop(0, _NLIST, drain, 0)

            def pass2(o, c1):
                rb = o * 128
                acc0 = jnp.zeros((16,), jnp.float32)
                acc1 = jnp.zeros((16,), jnp.float32)
                for c in range(8):
                    g0 = rows[pl.ds(rb + c * 16, 16)]
                    g1 = rows[pl.ds(half + rb + c * 16, 16)]
                    wv = wbuf[o, pl.ds(c * 16, 16)]
                    acc0 = acc0 + wv * g0
                    acc1 = acc1 + wv * g1
                pid = cbase + o * 16 + iota16
                oid = pid * _NOUT + (2 * lvl)
                plsc.store_scatter(outb, [oid], acc0 * 10.0)
                plsc.store_scatter(outb, [oid + 1], acc1 * 10.0)
                return c1

            lax.fori_loop(0, _NGRP, pass2, 0)
            return carry

        lax.fori_loop(0, _NCK, chunk_body, 0)

    pltpu.sync_copy(outb, out_ref.at[pl.ds(base * _NOUT, _PW * _NOUT)])


def _make_kernel():
    mesh = plsc.VectorSubcoreMesh(
        core_axis_name="c", subcore_axis_name="s",
        num_cores=_NC, num_subcores=_NS)
    return pl.kernel(
        _body,
        out_type=jax.ShapeDtypeStruct((_NPTS * _NOUT,), jnp.float32),
        mesh=mesh,
        compiler_params=pltpu.CompilerParams(
            needs_layout_passes=False, use_tc_tiling_on_sc=False),
        scratch_types=[
            pltpu.VMEM((_PW,), jnp.float32),
            pltpu.VMEM((_PW,), jnp.float32),
            pltpu.VMEM((_PW,), jnp.float32),
            pltpu.VMEM((_NGRP * 128,), jnp.int32),
            pltpu.VMEM((_NGRP, 128), jnp.float32),
            pltpu.VMEM((2 * _NGRP * 128,), jnp.float32),
            pltpu.VMEM((_PW * _NOUT,), jnp.float32),
            pltpu.SemaphoreType.DMA,
        ],
    )


def kernel(x, grid_values):
    tabs0 = []
    tabs1 = []
    for tv in grid_values:
        t2 = tv.reshape(-1, _NF)
        tabs0.append(t2[:, 0])
        tabs1.append(t2[:, 1])
    xt = x.T
    flat = _make_kernel()(xt, *tabs0, *tabs1)
    return flat.reshape(_NPTS, _NOUT)


# speedup vs baseline: 55.6627x; 1.0165x over previous
import jax
import jax.numpy as jnp
import numpy as np
from jax import lax
from jax.experimental import pallas as pl
from jax.experimental.pallas import tpu as pltpu
from jax.experimental.pallas import tpu_sc as plsc

_HASH_SIZE = 2 ** 19
_K1 = int(np.int32(np.uint32(2654435761)))
_K2 = int(np.int32(np.uint32(805459861)))
_NF = 2
_NPTS = 65536


def _level_grid_sizes():
    desired = 1.0 + 2.0 * np.log2(2048 / 16)
    n = int(np.round(desired))
    return [int(g) for g in np.round(np.geomspace(16, 2048, n)).astype(np.int32)]


_GS = _level_grid_sizes()
_IS_DENSE = [g ** 3 <= _HASH_SIZE for g in _GS]
_NLVL = len(_GS)
_NOUT = 2 * _NLVL

_NC, _NS = 2, 16
_NW = _NC * _NS
_PW = _NPTS // _NW
_CHUNK = 512
_NCK = _PW // _CHUNK
_NGRP = _CHUNK // 16
_LIST = 512
_GPL = _LIST // 128
_NLIST = _NGRP // _GPL
_HALFB = _NGRP * 128


def _body(x_ref, *rest):
    tabs0 = rest[:_NLVL]
    tabs1 = rest[_NLVL:2 * _NLVL]
    out_ref = rest[2 * _NLVL]
    (xbuf, ybuf, zbuf, idxb, wbuf, rows, outb, sem) = rest[2 * _NLVL + 1:]

    cid = lax.axis_index("c")
    sid = lax.axis_index("s")
    wid = sid * _NC + cid
    base = wid * _PW

    iota16 = lax.iota(jnp.int32, 16)

    pltpu.sync_copy(x_ref.at[0, pl.ds(base, _PW)], xbuf)
    pltpu.sync_copy(x_ref.at[1, pl.ds(base, _PW)], ybuf)
    pltpu.sync_copy(x_ref.at[2, pl.ds(base, _PW)], zbuf)

    def pass0(o, carry):
        s = pl.ds(o * 16, 16)
        xbuf[s] = (xbuf[s] + 2.0) * 0.25
        ybuf[s] = (ybuf[s] + 2.0) * 0.25
        zbuf[s] = (zbuf[s] + 2.0) * 0.25
        return carry

    lax.fori_loop(0, _PW // 16, pass0, 0)

    for lvl in range(_NLVL):
        gs = _GS[lvl]
        dense = _IS_DENSE[lvl]
        t0 = tabs0[lvl]
        t1 = tabs1[lvl]
        gsf = float(gs)

        def drain_pass2(qm, qbase, t0=t0, t1=t1, lvl=lvl):
            def drain(j, c1):
                s = pl.ds(j * _LIST, _LIST)
                pltpu.make_async_copy(t0.at[idxb.at[qm, s]],
                                      rows.at[qm, s], sem).wait()
                pltpu.make_async_copy(
                    t1.at[idxb.at[qm, s]],
                    rows.at[qm, pl.ds(_HALFB + j * _LIST, _LIST)], sem).wait()
                return c1

            lax.fori_loop(0, _NLIST, drain, 0)

            def pass2(o, c1):
                rb = o * 128
                acc0 = jnp.zeros((16,), jnp.float32)
                acc1 = jnp.zeros((16,), jnp.float32)
                for c in range(8):
                    g0 = rows[qm, pl.ds(rb + c * 16, 16)]
                    g1 = rows[qm, pl.ds(_HALFB + rb + c * 16, 16)]
                    wv = wbuf[qm, o, pl.ds(c * 16, 16)]
                    acc0 = acc0 + wv * g0
                    acc1 = acc1 + wv * g1
                pid = qbase + o * 16 + iota16
                oid = pid * _NOUT + (2 * lvl)
                plsc.store_scatter(outb, [oid], acc0 * 10.0)
                plsc.store_scatter(outb, [oid + 1], acc1 * 10.0)
                return c1

            lax.fori_loop(0, _NGRP, pass2, 0)

        def chunk_body(ck, carry, t0=t0, t1=t1, gs=gs, dense=dense,
                       lvl=lvl, gsf=gsf, drain_pass2=drain_pass2):
            pm = lax.rem(ck, 2)
            cbase = ck * _CHUNK

            def pass1(o, c1):
                p0 = cbase + o * 16
                vx = xbuf[pl.ds(p0, 16)] * gsf
                vy = ybuf[pl.ds(p0, 16)] * gsf
                vz = zbuf[pl.ds(p0, 16)] * gsf
                terms = []
                wpairs = []
                if dense:
                    for v, mulc in ((vz, gs * gs), (vy, gs), (vx, 1)):
                        lc = v - 0.5
                        it = lc.astype(jnp.int32)
                        ft = it.astype(jnp.float32)
                        it = jnp.where(ft > lc, it - 1, it)
                        ft = it.astype(jnp.float32)
                        t = lc - ft
                        i0 = jnp.maximum(it, 0) * mulc
                        i1 = jnp.minimum(it + 1, gs - 1) * mulc
                        terms.append((i0, i1))
                        wpairs.append((1.0 - t, t))
                else:
                    for v, kc in ((vx, 1), (vy, _K1), (vz, _K2)):
                        iv = v.astype(jnp.int32)
                        fv = iv.astype(jnp.float32)
                        t = v - fv
                        h0 = iv * kc if kc != 1 else iv
                        h1 = h0 + kc
                        terms.append((h0, h1))
                        wpairs.append((1.0 - t, t))
                for c in range(8):
                    a, b, d = (c >> 2) & 1, (c >> 1) & 1, c & 1
                    if dense:
                        row = terms[0][a] + terms[1][b] + terms[2][d]
                    else:
                        row = ((terms[0][a] ^ terms[1][b]) ^ terms[2][d]) & (_HASH_SIZE - 1)
                    w = (wpairs[0][a] * wpairs[1][b]) * wpairs[2][d]
                    idxb[pm, pl.ds(o * 128 + c * 16, 16)] = row
                    wbuf[pm, o, pl.ds(c * 16, 16)] = w

                @pl.when(lax.rem(o, _GPL) == _GPL - 1)
                def _():
                    j = lax.div(o, _GPL)
                    s = pl.ds(j * _LIST, _LIST)
                    pltpu.make_async_copy(t0.at[idxb.at[pm, s]],
                                          rows.at[pm, s], sem).start()
                    pltpu.make_async_copy(
                        t1.at[idxb.at[pm, s]],
                        rows.at[pm, pl.ds(_HALFB + j * _LIST, _LIST)],
                        sem).start()
                return c1

            lax.fori_loop(0, _NGRP, pass1, 0)

            @pl.when(ck > 0)
            def _():
                drain_pass2(1 - pm, cbase - _CHUNK)
            return carry

        lax.fori_loop(0, _NCK, chunk_body, 0)
        drain_pass2((_NCK - 1) % 2, (_NCK - 1) * _CHUNK)

    pltpu.sync_copy(outb, out_ref.at[pl.ds(base * _NOUT, _PW * _NOUT)])


def _make_kernel():
    mesh = plsc.VectorSubcoreMesh(
        core_axis_name="c", subcore_axis_name="s",
        num_cores=_NC, num_subcores=_NS)
    return pl.kernel(
        _body,
        out_type=jax.ShapeDtypeStruct((_NPTS * _NOUT,), jnp.float32),
        mesh=mesh,
        compiler_params=pltpu.CompilerParams(
            needs_layout_passes=False, use_tc_tiling_on_sc=False),
        scratch_types=[
            pltpu.VMEM((_PW,), jnp.float32),
            pltpu.VMEM((_PW,), jnp.float32),
            pltpu.VMEM((_PW,), jnp.float32),
            pltpu.VMEM((2, _NGRP * 128), jnp.int32),
            pltpu.VMEM((2, _NGRP, 128), jnp.float32),
            pltpu.VMEM((2, 2 * _NGRP * 128), jnp.float32),
            pltpu.VMEM((_PW * _NOUT,), jnp.float32),
            pltpu.SemaphoreType.DMA,
        ],
    )


def kernel(x, grid_values):
    tabs0 = []
    tabs1 = []
    for tv in grid_values:
        t2 = tv.reshape(-1, _NF)
        tabs0.append(t2[:, 0])
        tabs1.append(t2[:, 1])
    xt = x.T
    flat = _make_kernel()(xt, *tabs0, *tabs1)
    return flat.reshape(_NPTS, _NOUT)


# speedup vs baseline: 60.9347x; 1.0947x over previous
import jax
import jax.numpy as jnp
import numpy as np
from jax import lax
from jax.experimental import pallas as pl
from jax.experimental.pallas import tpu as pltpu
from jax.experimental.pallas import tpu_sc as plsc

_HASH_SIZE = 2 ** 19
_K1 = int(np.int32(np.uint32(2654435761)))
_K2 = int(np.int32(np.uint32(805459861)))
_NF = 2
_NPTS = 65536


def _level_grid_sizes():
    desired = 1.0 + 2.0 * np.log2(2048 / 16)
    n = int(np.round(desired))
    return [int(g) for g in np.round(np.geomspace(16, 2048, n)).astype(np.int32)]


_GS = _level_grid_sizes()
_IS_DENSE = [g ** 3 <= _HASH_SIZE for g in _GS]
_NLVL = len(_GS)
_NOUT = 2 * _NLVL

_NC, _NS = 2, 16
_NW = _NC * _NS
_PW = _NPTS // _NW
_CHUNK = 512
_NCK = _PW // _CHUNK
_NGRP = _CHUNK // 16
_LIST = 1024
_GPL = _LIST // 128
_NLIST = _NGRP // _GPL
_HALFB = _NGRP * 128


def _body(x_ref, *rest):
    tabs0 = rest[:_NLVL]
    tabs1 = rest[_NLVL:2 * _NLVL]
    out_ref = rest[2 * _NLVL]
    (xbuf, ybuf, zbuf, idxb, wbuf, rows, outb, lt0, lt1, sem) = rest[2 * _NLVL + 1:]

    cid = lax.axis_index("c")
    sid = lax.axis_index("s")
    wid = sid * _NC + cid
    base = wid * _PW

    iota16 = lax.iota(jnp.int32, 16)

    pltpu.sync_copy(x_ref.at[0, pl.ds(base, _PW)], xbuf)
    pltpu.sync_copy(x_ref.at[1, pl.ds(base, _PW)], ybuf)
    pltpu.sync_copy(x_ref.at[2, pl.ds(base, _PW)], zbuf)

    def pass0(o, carry):
        s = pl.ds(o * 16, 16)
        xbuf[s] = (xbuf[s] + 2.0) * 0.25
        ybuf[s] = (ybuf[s] + 2.0) * 0.25
        zbuf[s] = (zbuf[s] + 2.0) * 0.25
        return carry

    lax.fori_loop(0, _PW // 16, pass0, 0)

    pltpu.sync_copy(tabs0[0], lt0)
    pltpu.sync_copy(tabs1[0], lt1)

    g0s = _GS[0]
    g0f = float(g0s)

    def lvl0(o, carry):
        vx = xbuf[pl.ds(o * 16, 16)] * g0f
        vy = ybuf[pl.ds(o * 16, 16)] * g0f
        vz = zbuf[pl.ds(o * 16, 16)] * g0f
        terms = []
        wpairs = []
        for v, mulc in ((vz, g0s * g0s), (vy, g0s), (vx, 1)):
            lc = v - 0.5
            it = lc.astype(jnp.int32)
            ft = it.astype(jnp.float32)
            it = jnp.where(ft > lc, it - 1, it)
            ft = it.astype(jnp.float32)
            t = lc - ft
            i0 = jnp.maximum(it, 0) * mulc
            i1 = jnp.minimum(it + 1, g0s - 1) * mulc
            terms.append((i0, i1))
            wpairs.append((1.0 - t, t))
        acc0 = jnp.zeros((16,), jnp.float32)
        acc1 = jnp.zeros((16,), jnp.float32)
        for c in range(8):
            a, b, d = (c >> 2) & 1, (c >> 1) & 1, c & 1
            row = terms[0][a] + terms[1][b] + terms[2][d]
            w = (wpairs[0][a] * wpairs[1][b]) * wpairs[2][d]
            acc0 = acc0 + w * plsc.load_gather(lt0, [row])
            acc1 = acc1 + w * plsc.load_gather(lt1, [row])
        pid = o * 16 + iota16
        oid = pid * _NOUT
        plsc.store_scatter(outb, [oid], acc0 * 10.0)
        plsc.store_scatter(outb, [oid + 1], acc1 * 10.0)
        return carry

    lax.fori_loop(0, _PW // 16, lvl0, 0)

    for lvl in range(1, _NLVL):
        gs = _GS[lvl]
        dense = _IS_DENSE[lvl]
        t0 = tabs0[lvl]
        t1 = tabs1[lvl]
        gsf = float(gs)

        def drain_pass2(qm, qbase, t0=t0, t1=t1, lvl=lvl):
            def drain(j, c1):
                s = pl.ds(j * _LIST, _LIST)
                pltpu.make_async_copy(t0.at[idxb.at[qm, s]],
                                      rows.at[qm, s], sem).wait()
                pltpu.make_async_copy(
                    t1.at[idxb.at[qm, s]],
                    rows.at[qm, pl.ds(_HALFB + j * _LIST, _LIST)], sem).wait()
                return c1

            lax.fori_loop(0, _NLIST, drain, 0)

            def pass2(o, c1):
                rb = o * 128
                acc0 = jnp.zeros((16,), jnp.float32)
                acc1 = jnp.zeros((16,), jnp.float32)
                for c in range(8):
                    g0 = rows[qm, pl.ds(rb + c * 16, 16)]
                    g1 = rows[qm, pl.ds(_HALFB + rb + c * 16, 16)]
                    wv = wbuf[qm, o, pl.ds(c * 16, 16)]
                    acc0 = acc0 + wv * g0
                    acc1 = acc1 + wv * g1
                pid = qbase + o * 16 + iota16
                oid = pid * _NOUT + (2 * lvl)
                plsc.store_scatter(outb, [oid], acc0 * 10.0)
                plsc.store_scatter(outb, [oid + 1], acc1 * 10.0)
                return c1

            lax.fori_loop(0, _NGRP, pass2, 0)

        def chunk_body(ck, carry, t0=t0, t1=t1, gs=gs, dense=dense,
                       lvl=lvl, gsf=gsf, drain_pass2=drain_pass2):
            pm = lax.rem(ck, 2)
            cbase = ck * _CHUNK

            def pass1(o, c1):
                p0 = cbase + o * 16
                vx = xbuf[pl.ds(p0, 16)] * gsf
                vy = ybuf[pl.ds(p0, 16)] * gsf
                vz = zbuf[pl.ds(p0, 16)] * gsf
                terms = []
                wpairs = []
                if dense:
                    for v, mulc in ((vz, gs * gs), (vy, gs), (vx, 1)):
                        lc = v - 0.5
                        it = lc.astype(jnp.int32)
                        ft = it.astype(jnp.float32)
                        it = jnp.where(ft > lc, it - 1, it)
                        ft = it.astype(jnp.float32)
                        t = lc - ft
                        i0 = jnp.maximum(it, 0) * mulc
                        i1 = jnp.minimum(it + 1, gs - 1) * mulc
                        terms.append((i0, i1))
                        wpairs.append((1.0 - t, t))
                else:
                    for v, kc in ((vx, 1), (vy, _K1), (vz, _K2)):
                        iv = v.astype(jnp.int32)
                        fv = iv.astype(jnp.float32)
                        t = v - fv
                        h0 = iv * kc if kc != 1 else iv
                        h1 = h0 + kc
                        terms.append((h0, h1))
                        wpairs.append((1.0 - t, t))
                for c in range(8):
                    a, b, d = (c >> 2) & 1, (c >> 1) & 1, c & 1
                    if dense:
                        row = terms[0][a] + terms[1][b] + terms[2][d]
                    else:
                        row = ((terms[0][a] ^ terms[1][b]) ^ terms[2][d]) & (_HASH_SIZE - 1)
                    w = (wpairs[0][a] * wpairs[1][b]) * wpairs[2][d]
                    idxb[pm, pl.ds(o * 128 + c * 16, 16)] = row
                    wbuf[pm, o, pl.ds(c * 16, 16)] = w

                @pl.when(lax.rem(o, _GPL) == _GPL - 1)
                def _():
                    j = lax.div(o, _GPL)
                    s = pl.ds(j * _LIST, _LIST)
                    pltpu.make_async_copy(t0.at[idxb.at[pm, s]],
                                          rows.at[pm, s], sem).start()
                    pltpu.make_async_copy(
                        t1.at[idxb.at[pm, s]],
                        rows.at[pm, pl.ds(_HALFB + j * _LIST, _LIST)],
                        sem).start()
                return c1

            lax.fori_loop(0, _NGRP, pass1, 0)

            @pl.when(ck > 0)
            def _():
                drain_pass2(1 - pm, cbase - _CHUNK)
            return carry

        lax.fori_loop(0, _NCK, chunk_body, 0)
        drain_pass2((_NCK - 1) % 2, (_NCK - 1) * _CHUNK)

    pltpu.sync_copy(outb, out_ref.at[pl.ds(base * _NOUT, _PW * _NOUT)])


def _make_kernel():
    mesh = plsc.VectorSubcoreMesh(
        core_axis_name="c", subcore_axis_name="s",
        num_cores=_NC, num_subcores=_NS)
    return pl.kernel(
        _body,
        out_type=jax.ShapeDtypeStruct((_NPTS * _NOUT,), jnp.float32),
        mesh=mesh,
        compiler_params=pltpu.CompilerParams(
            needs_layout_passes=False, use_tc_tiling_on_sc=False),
        scratch_types=[
            pltpu.VMEM((_PW,), jnp.float32),
            pltpu.VMEM((_PW,), jnp.float32),
            pltpu.VMEM((_PW,), jnp.float32),
            pltpu.VMEM((2, _NGRP * 128), jnp.int32),
            pltpu.VMEM((2, _NGRP, 128), jnp.float32),
            pltpu.VMEM((2, 2 * _NGRP * 128), jnp.float32),
            pltpu.VMEM((_PW * _NOUT,), jnp.float32),
            pltpu.VMEM((_GS[0] ** 3,), jnp.float32),
            pltpu.VMEM((_GS[0] ** 3,), jnp.float32),
            pltpu.SemaphoreType.DMA,
        ],
    )


def kernel(x, grid_values):
    tabs0 = []
    tabs1 = []
    for tv in grid_values:
        t2 = tv.reshape(-1, _NF)
        tabs0.append(t2[:, 0])
        tabs1.append(t2[:, 1])
    xt = x.T
    flat = _make_kernel()(xt, *tabs0, *tabs1)
    return flat.reshape(_NPTS, _NOUT)


# speedup vs baseline: 64.1669x; 1.0530x over previous
import jax
import jax.numpy as jnp
import numpy as np
from jax import lax
from jax.experimental import pallas as pl
from jax.experimental.pallas import tpu as pltpu
from jax.experimental.pallas import tpu_sc as plsc

_HASH_SIZE = 2 ** 19
_K1 = int(np.int32(np.uint32(2654435761)))
_K2 = int(np.int32(np.uint32(805459861)))
_NF = 2
_NPTS = 65536


def _level_grid_sizes():
    desired = 1.0 + 2.0 * np.log2(2048 / 16)
    n = int(np.round(desired))
    return [int(g) for g in np.round(np.geomspace(16, 2048, n)).astype(np.int32)]


_GS = _level_grid_sizes()
_IS_DENSE = [g ** 3 <= _HASH_SIZE for g in _GS]
_NLVL = len(_GS)
_NOUT = 2 * _NLVL

_NC, _NS = 2, 16
_NW = _NC * _NS
_PW = _NPTS // _NW
_CHUNK = 256
_NCK = _PW // _CHUNK
_NGRP = _CHUNK // 16
_LIST = 1024
_GPL = _LIST // 128
_NLIST = _NGRP // _GPL
_HALFB = _NGRP * 128


def _body(x_ref, *rest):
    tabs0 = rest[:_NLVL]
    tabs1 = rest[_NLVL:2 * _NLVL]
    out_ref = rest[2 * _NLVL]
    (xbuf, ybuf, zbuf, idxb, wbuf, rows, outb, lt0a, lt1a, lt0b, lt1b, sem) = rest[2 * _NLVL + 1:]

    cid = lax.axis_index("c")
    sid = lax.axis_index("s")
    wid = sid * _NC + cid
    base = wid * _PW

    iota16 = lax.iota(jnp.int32, 16)

    pltpu.sync_copy(x_ref.at[0, pl.ds(base, _PW)], xbuf)
    pltpu.sync_copy(x_ref.at[1, pl.ds(base, _PW)], ybuf)
    pltpu.sync_copy(x_ref.at[2, pl.ds(base, _PW)], zbuf)

    def pass0(o, carry):
        s = pl.ds(o * 16, 16)
        xbuf[s] = (xbuf[s] + 2.0) * 0.25
        ybuf[s] = (ybuf[s] + 2.0) * 0.25
        zbuf[s] = (zbuf[s] + 2.0) * 0.25
        return carry

    lax.fori_loop(0, _PW // 16, pass0, 0)

    pltpu.sync_copy(tabs0[0], lt0a)
    pltpu.sync_copy(tabs1[0], lt1a)
    pltpu.sync_copy(tabs0[1], lt0b)
    pltpu.sync_copy(tabs1[1], lt1b)

    for llvl, l0, l1 in ((0, lt0a, lt1a), (1, lt0b, lt1b)):
        gls = _GS[llvl]
        glf = float(gls)

        def lvl_local(o, carry, gls=gls, glf=glf, l0=l0, l1=l1, llvl=llvl):
            vx = xbuf[pl.ds(o * 16, 16)] * glf
            vy = ybuf[pl.ds(o * 16, 16)] * glf
            vz = zbuf[pl.ds(o * 16, 16)] * glf
            terms = []
            wpairs = []
            for v, mulc in ((vz, gls * gls), (vy, gls), (vx, 1)):
                lc = v - 0.5
                it = lc.astype(jnp.int32)
                ft = it.astype(jnp.float32)
                it = jnp.where(ft > lc, it - 1, it)
                ft = it.astype(jnp.float32)
                t = lc - ft
                i0 = jnp.maximum(it, 0) * mulc
                i1 = jnp.minimum(it + 1, gls - 1) * mulc
                terms.append((i0, i1))
                wpairs.append((1.0 - t, t))
            acc0 = jnp.zeros((16,), jnp.float32)
            acc1 = jnp.zeros((16,), jnp.float32)
            for c in range(8):
                a, b, d = (c >> 2) & 1, (c >> 1) & 1, c & 1
                row = terms[0][a] + terms[1][b] + terms[2][d]
                w = (wpairs[0][a] * wpairs[1][b]) * wpairs[2][d]
                acc0 = acc0 + w * plsc.load_gather(l0, [row])
                acc1 = acc1 + w * plsc.load_gather(l1, [row])
            pid = o * 16 + iota16
            oid = pid * _NOUT + (2 * llvl)
            plsc.store_scatter(outb, [oid], acc0 * 10.0)
            plsc.store_scatter(outb, [oid + 1], acc1 * 10.0)
            return carry

        lax.fori_loop(0, _PW // 16, lvl_local, 0)

    for lvl in range(2, _NLVL):
        gs = _GS[lvl]
        dense = _IS_DENSE[lvl]
        t0 = tabs0[lvl]
        t1 = tabs1[lvl]
        gsf = float(gs)

        def drain_pass2(qm, qbase, t0=t0, t1=t1, lvl=lvl):
            def drain(j, c1):
                s = pl.ds(j * _LIST, _LIST)
                pltpu.make_async_copy(t0.at[idxb.at[qm, s]],
                                      rows.at[qm, s], sem).wait()
                pltpu.make_async_copy(
                    t1.at[idxb.at[qm, s]],
                    rows.at[qm, pl.ds(_HALFB + j * _LIST, _LIST)], sem).wait()
                return c1

            lax.fori_loop(0, _NLIST, drain, 0)

            def pass2(o, c1):
                rb = o * 128
                acc0 = jnp.zeros((16,), jnp.float32)
                acc1 = jnp.zeros((16,), jnp.float32)
                for c in range(8):
                    g0 = rows[qm, pl.ds(rb + c * 16, 16)]
                    g1 = rows[qm, pl.ds(_HALFB + rb + c * 16, 16)]
                    wv = wbuf[qm, o, pl.ds(c * 16, 16)]
                    acc0 = acc0 + wv * g0
                    acc1 = acc1 + wv * g1
                pid = qbase + o * 16 + iota16
                oid = pid * _NOUT + (2 * lvl)
                plsc.store_scatter(outb, [oid], acc0 * 10.0)
                plsc.store_scatter(outb, [oid + 1], acc1 * 10.0)
                return c1

            lax.fori_loop(0, _NGRP, pass2, 0)

        def chunk_body(ck, carry, t0=t0, t1=t1, gs=gs, dense=dense,
                       lvl=lvl, gsf=gsf, drain_pass2=drain_pass2):
            pm = lax.rem(ck, 2)
            cbase = ck * _CHUNK

            def pass1(o, c1):
                p0 = cbase + o * 16
                vx = xbuf[pl.ds(p0, 16)] * gsf
                vy = ybuf[pl.ds(p0, 16)] * gsf
                vz = zbuf[pl.ds(p0, 16)] * gsf
                terms = []
                wpairs = []
                if dense:
                    for v, mulc in ((vz, gs * gs), (vy, gs), (vx, 1)):
                        lc = v - 0.5
                        it = lc.astype(jnp.int32)
                        ft = it.astype(jnp.float32)
                        it = jnp.where(ft > lc, it - 1, it)
                        ft = it.astype(jnp.float32)
                        t = lc - ft
                        i0 = jnp.maximum(it, 0) * mulc
                        i1 = jnp.minimum(it + 1, gs - 1) * mulc
                        terms.append((i0, i1))
                        wpairs.append((1.0 - t, t))
                else:
                    for v, kc in ((vx, 1), (vy, _K1), (vz, _K2)):
                        iv = v.astype(jnp.int32)
                        fv = iv.astype(jnp.float32)
                        t = v - fv
                        h0 = iv * kc if kc != 1 else iv
                        h1 = h0 + kc
                        terms.append((h0, h1))
                        wpairs.append((1.0 - t, t))
                for c in range(8):
                    a, b, d = (c >> 2) & 1, (c >> 1) & 1, c & 1
                    if dense:
                        row = terms[0][a] + terms[1][b] + terms[2][d]
                    else:
                        row = ((terms[0][a] ^ terms[1][b]) ^ terms[2][d]) & (_HASH_SIZE - 1)
                    w = (wpairs[0][a] * wpairs[1][b]) * wpairs[2][d]
                    idxb[pm, pl.ds(o * 128 + c * 16, 16)] = row
                    wbuf[pm, o, pl.ds(c * 16, 16)] = w

                @pl.when(lax.rem(o, _GPL) == _GPL - 1)
                def _():
                    j = lax.div(o, _GPL)
                    s = pl.ds(j * _LIST, _LIST)
                    pltpu.make_async_copy(t0.at[idxb.at[pm, s]],
                                          rows.at[pm, s], sem).start()
                    pltpu.make_async_copy(
                        t1.at[idxb.at[pm, s]],
                        rows.at[pm, pl.ds(_HALFB + j * _LIST, _LIST)],
                        sem).start()
                return c1

            lax.fori_loop(0, _NGRP, pass1, 0)

            @pl.when(ck > 0)
            def _():
                drain_pass2(1 - pm, cbase - _CHUNK)
            return carry

        lax.fori_loop(0, _NCK, chunk_body, 0)
        drain_pass2((_NCK - 1) % 2, (_NCK - 1) * _CHUNK)

    pltpu.sync_copy(outb, out_ref.at[pl.ds(base * _NOUT, _PW * _NOUT)])


def _make_kernel():
    mesh = plsc.VectorSubcoreMesh(
        core_axis_name="c", subcore_axis_name="s",
        num_cores=_NC, num_subcores=_NS)
    return pl.kernel(
        _body,
        out_type=jax.ShapeDtypeStruct((_NPTS * _NOUT,), jnp.float32),
        mesh=mesh,
        compiler_params=pltpu.CompilerParams(
            needs_layout_passes=False, use_tc_tiling_on_sc=False),
        scratch_types=[
            pltpu.VMEM((_PW,), jnp.float32),
            pltpu.VMEM((_PW,), jnp.float32),
            pltpu.VMEM((_PW,), jnp.float32),
            pltpu.VMEM((2, _NGRP * 128), jnp.int32),
            pltpu.VMEM((2, _NGRP, 128), jnp.float32),
            pltpu.VMEM((2, 2 * _NGRP * 128), jnp.float32),
            pltpu.VMEM((_PW * _NOUT,), jnp.float32),
            pltpu.VMEM((_GS[0] ** 3,), jnp.float32),
            pltpu.VMEM((_GS[0] ** 3,), jnp.float32),
            pltpu.VMEM((_GS[1] ** 3,), jnp.float32),
            pltpu.VMEM((_GS[1] ** 3,), jnp.float32),
            pltpu.SemaphoreType.DMA,
        ],
    )


def kernel(x, grid_values):
    tabs0 = []
    tabs1 = []
    for tv in grid_values:
        t2 = tv.reshape(-1, _NF)
        tabs0.append(t2[:, 0])
        tabs1.append(t2[:, 1])
    xt = x.T
    flat = _make_kernel()(xt, *tabs0, *tabs1)
    return flat.reshape(_NPTS, _NOUT)


# speedup vs baseline: 64.7278x; 1.0087x over previous
import jax
import jax.numpy as jnp
import numpy as np
from jax import lax
from jax.experimental import pallas as pl
from jax.experimental.pallas import tpu as pltpu
from jax.experimental.pallas import tpu_sc as plsc

_HASH_SIZE = 2 ** 19
_K1 = int(np.int32(np.uint32(2654435761)))
_K2 = int(np.int32(np.uint32(805459861)))
_NF = 2
_NPTS = 65536


def _level_grid_sizes():
    desired = 1.0 + 2.0 * np.log2(2048 / 16)
    n = int(np.round(desired))
    return [int(g) for g in np.round(np.geomspace(16, 2048, n)).astype(np.int32)]


_GS = _level_grid_sizes()
_IS_DENSE = [g ** 3 <= _HASH_SIZE for g in _GS]
_NLVL = len(_GS)
_NOUT = 2 * _NLVL

_NC, _NS = 2, 16
_NW = _NC * _NS
_PW = _NPTS // _NW
_CHUNK = 256
_NCK = _PW // _CHUNK
_NGRP = _CHUNK // 16
_LIST = 1024
_GPL = _LIST // 128
_NLIST = _NGRP // _GPL
_HALFB = _NGRP * 128


def _body(x_ref, *rest):
    tabs0 = rest[:_NLVL]
    tabs1 = rest[_NLVL:2 * _NLVL]
    out_ref = rest[2 * _NLVL]
    (xbuf, ybuf, zbuf, idxb, wbuf, rows, outb, lt0a, lt1a, lt0b, lt1b, sem) = rest[2 * _NLVL + 1:]

    cid = lax.axis_index("c")
    sid = lax.axis_index("s")
    wid = sid * _NC + cid
    base = wid * _PW

    iota16 = lax.iota(jnp.int32, 16)

    pltpu.sync_copy(x_ref.at[0, pl.ds(base, _PW)], xbuf)
    pltpu.sync_copy(x_ref.at[1, pl.ds(base, _PW)], ybuf)
    pltpu.sync_copy(x_ref.at[2, pl.ds(base, _PW)], zbuf)

    def pass0(o, carry):
        s = pl.ds(o * 16, 16)
        xbuf[s] = (xbuf[s] + 2.0) * 0.25
        ybuf[s] = (ybuf[s] + 2.0) * 0.25
        zbuf[s] = (zbuf[s] + 2.0) * 0.25
        return carry

    lax.fori_loop(0, _PW // 16, pass0, 0)

    pltpu.sync_copy(tabs0[0], lt0a)
    pltpu.sync_copy(tabs1[0], lt1a)
    pltpu.sync_copy(tabs0[1], lt0b)
    pltpu.sync_copy(tabs1[1], lt1b)

    for llvl, l0, l1 in ((0, lt0a, lt1a), (1, lt0b, lt1b)):
        gls = _GS[llvl]
        glf = float(gls)

        def lvl_local(o, carry, gls=gls, glf=glf, l0=l0, l1=l1, llvl=llvl):
            vx = xbuf[pl.ds(o * 16, 16)] * glf
            vy = ybuf[pl.ds(o * 16, 16)] * glf
            vz = zbuf[pl.ds(o * 16, 16)] * glf
            terms = []
            wpairs = []
            for v, mulc in ((vz, gls * gls), (vy, gls), (vx, 1)):
                lc = v - 0.5
                it = lc.astype(jnp.int32)
                ft = it.astype(jnp.float32)
                it = jnp.where(ft > lc, it - 1, it)
                ft = it.astype(jnp.float32)
                t = lc - ft
                i0 = jnp.maximum(it, 0) * mulc
                i1 = jnp.minimum(it + 1, gls - 1) * mulc
                terms.append((i0, i1))
                wpairs.append((1.0 - t, t))
            acc0 = jnp.zeros((16,), jnp.float32)
            acc1 = jnp.zeros((16,), jnp.float32)
            for c in range(8):
                a, b, d = (c >> 2) & 1, (c >> 1) & 1, c & 1
                row = terms[0][a] + terms[1][b] + terms[2][d]
                w = (wpairs[0][a] * wpairs[1][b]) * wpairs[2][d]
                acc0 = acc0 + w * plsc.load_gather(l0, [row])
                acc1 = acc1 + w * plsc.load_gather(l1, [row])
            pid = o * 16 + iota16
            plsc.store_scatter(outb, [pid, jnp.full((16,), 2 * llvl, jnp.int32)], acc0 * 10.0)
            plsc.store_scatter(outb, [pid, jnp.full((16,), 2 * llvl + 1, jnp.int32)], acc1 * 10.0)
            return carry

        lax.fori_loop(0, _PW // 16, lvl_local, 0)

    for lvl in range(2, _NLVL):
        gs = _GS[lvl]
        dense = _IS_DENSE[lvl]
        t0 = tabs0[lvl]
        t1 = tabs1[lvl]
        gsf = float(gs)

        def drain_pass2(qm, qbase, t0=t0, t1=t1, lvl=lvl):
            def drain(j, c1):
                s = pl.ds(j * _LIST, _LIST)
                pltpu.make_async_copy(t0.at[idxb.at[qm, s]],
                                      rows.at[qm, s], sem).wait()
                pltpu.make_async_copy(
                    t1.at[idxb.at[qm, s]],
                    rows.at[qm, pl.ds(_HALFB + j * _LIST, _LIST)], sem).wait()
                return c1

            lax.fori_loop(0, _NLIST, drain, 0)

            def pass2(o, c1):
                rb = o * 128
                acc0 = jnp.zeros((16,), jnp.float32)
                acc1 = jnp.zeros((16,), jnp.float32)
                for c in range(8):
                    g0 = rows[qm, pl.ds(rb + c * 16, 16)]
                    g1 = rows[qm, pl.ds(_HALFB + rb + c * 16, 16)]
                    wv = wbuf[qm, o, pl.ds(c * 16, 16)]
                    acc0 = acc0 + wv * g0
                    acc1 = acc1 + wv * g1
                pid = qbase + o * 16 + iota16
                plsc.store_scatter(outb, [pid, jnp.full((16,), 2 * lvl, jnp.int32)], acc0 * 10.0)
                plsc.store_scatter(outb, [pid, jnp.full((16,), 2 * lvl + 1, jnp.int32)], acc1 * 10.0)
                return c1

            lax.fori_loop(0, _NGRP, pass2, 0)

        def chunk_body(ck, carry, t0=t0, t1=t1, gs=gs, dense=dense,
                       lvl=lvl, gsf=gsf, drain_pass2=drain_pass2):
            pm = lax.rem(ck, 2)
            cbase = ck * _CHUNK

            def pass1(o, c1):
                p0 = cbase + o * 16
                vx = xbuf[pl.ds(p0, 16)] * gsf
                vy = ybuf[pl.ds(p0, 16)] * gsf
                vz = zbuf[pl.ds(p0, 16)] * gsf
                terms = []
                wpairs = []
                if dense:
                    for v, mulc in ((vz, gs * gs), (vy, gs), (vx, 1)):
                        lc = v - 0.5
                        it = lc.astype(jnp.int32)
                        ft = it.astype(jnp.float32)
                        it = jnp.where(ft > lc, it - 1, it)
                        ft = it.astype(jnp.float32)
                        t = lc - ft
                        i0 = jnp.maximum(it, 0) * mulc
                        i1 = jnp.minimum(it + 1, gs - 1) * mulc
                        terms.append((i0, i1))
                        wpairs.append((1.0 - t, t))
                else:
                    for v, kc in ((vx, 1), (vy, _K1), (vz, _K2)):
                        iv = v.astype(jnp.int32)
                        fv = iv.astype(jnp.float32)
                        t = v - fv
                        h0 = iv * kc if kc != 1 else iv
                        h1 = h0 + kc
                        terms.append((h0, h1))
                        wpairs.append((1.0 - t, t))
                for c in range(8):
                    a, b, d = (c >> 2) & 1, (c >> 1) & 1, c & 1
                    if dense:
                        row = terms[0][a] + terms[1][b] + terms[2][d]
                    else:
                        row = ((terms[0][a] ^ terms[1][b]) ^ terms[2][d]) & (_HASH_SIZE - 1)
                    w = (wpairs[0][a] * wpairs[1][b]) * wpairs[2][d]
                    idxb[pm, pl.ds(o * 128 + c * 16, 16)] = row
                    wbuf[pm, o, pl.ds(c * 16, 16)] = w

                @pl.when(lax.rem(o, _GPL) == _GPL - 1)
                def _():
                    j = lax.div(o, _GPL)
                    s = pl.ds(j * _LIST, _LIST)
                    pltpu.make_async_copy(t0.at[idxb.at[pm, s]],
                                          rows.at[pm, s], sem).start()
                    pltpu.make_async_copy(
                        t1.at[idxb.at[pm, s]],
                        rows.at[pm, pl.ds(_HALFB + j * _LIST, _LIST)],
                        sem).start()
                return c1

            lax.fori_loop(0, _NGRP, pass1, 0)

            @pl.when(ck > 0)
            def _():
                drain_pass2(1 - pm, cbase - _CHUNK)
            return carry

        lax.fori_loop(0, _NCK, chunk_body, 0)
        drain_pass2((_NCK - 1) % 2, (_NCK - 1) * _CHUNK)

    pltpu.sync_copy(outb, out_ref.at[pl.ds(base, _PW), :])


def _make_kernel():
    mesh = plsc.VectorSubcoreMesh(
        core_axis_name="c", subcore_axis_name="s",
        num_cores=_NC, num_subcores=_NS)
    return pl.kernel(
        _body,
        out_type=jax.ShapeDtypeStruct((_NPTS, _NOUT), jnp.float32),
        mesh=mesh,
        compiler_params=pltpu.CompilerParams(
            needs_layout_passes=False, use_tc_tiling_on_sc=False),
        scratch_types=[
            pltpu.VMEM((_PW,), jnp.float32),
            pltpu.VMEM((_PW,), jnp.float32),
            pltpu.VMEM((_PW,), jnp.float32),
            pltpu.VMEM((2, _NGRP * 128), jnp.int32),
            pltpu.VMEM((2, _NGRP, 128), jnp.float32),
            pltpu.VMEM((2, 2 * _NGRP * 128), jnp.float32),
            pltpu.VMEM((_PW, _NOUT), jnp.float32),
            pltpu.VMEM((_GS[0] ** 3,), jnp.float32),
            pltpu.VMEM((_GS[0] ** 3,), jnp.float32),
            pltpu.VMEM((_GS[1] ** 3,), jnp.float32),
            pltpu.VMEM((_GS[1] ** 3,), jnp.float32),
            pltpu.SemaphoreType.DMA,
        ],
    )


def kernel(x, grid_values):
    tabs0 = []
    tabs1 = []
    for tv in grid_values:
        t2 = tv.reshape(-1, _NF)
        tabs0.append(t2[:, 0])
        tabs1.append(t2[:, 1])
    xt = x.T
    return _make_kernel()(xt, *tabs0, *tabs1)
